# Initial kernel scaffold; baseline (speedup 1.0000x reference)
import jax
import jax.numpy as jnp
from jax.experimental import pallas as pl


def kernel(X, target, proj_args):
    raise NotImplementedError("write your pallas kernel here")


# speedup vs baseline: 8.6283x; 8.6283x over previous
import functools

import jax
import jax.numpy as jnp
from jax import lax
from jax.experimental import pallas as pl
from jax.experimental.pallas import tpu as pltpu
from jax.experimental.pallas import tpu_sc as plsc

N = 128
D = 32768
L = 16
NC = 2
NS = 16
NW = NC * NS
RPW = N // NW
NV = D // L
NEG = -3.0e38
TCAP = 256

_f32 = jnp.float32
_i32 = jnp.int32


def _wid():
    return lax.axis_index("s") * NC + lax.axis_index("c")


def _negvec():
    return jnp.full((L,), NEG, _f32)


def _collect(row_v, cand_v, thr):
    def body(i, off):
        v = row_v[pl.ds(i * L, L)]
        mask = v > thr
        plsc.store_compressed(cand_v.at[pl.ds(off, L)], v, mask=mask)
        return off + jnp.sum(mask.astype(_i32))
    c = lax.fori_loop(0, NV, body, _i32(0))
    cand_v[pl.ds(c, L)] = _negvec()
    return c


def _sweep_max(cand_v, c):
    nvv = lax.shift_right_arithmetic(c + (L - 1), 4)
    def body(i, a):
        return jnp.maximum(a, cand_v[pl.ds(i * L, L)])
    a = lax.fori_loop(0, nvv, body, _negvec())
    return jnp.max(a)


def _sweep_count_remove(cand_v, c, vs):
    nvv = lax.shift_right_arithmetic(c + (L - 1), 4)
    def body(i, mu):
        v = cand_v[pl.ds(i * L, L)]
        eq = v == vs
        cand_v[pl.ds(i * L, L)] = jnp.where(eq, _negvec(), v)
        return mu + jnp.sum(eq.astype(_i32))
    return lax.fori_loop(0, nvv, body, _i32(0))


def _stage1_body(x_hbm, tgt_hbm, tau_hbm, k_hbm, m_hbm, xt_hbm,
                 row_v, cand_v, tgt_v, tau_s, k_s, m_s, xt_s):
    w = _wid()
    pltpu.sync_copy(tgt_hbm, tgt_v)
    lane = lax.iota(_i32, L)
    tau_acc = jnp.zeros((L,), _f32)
    k_acc = jnp.full((L,), 1, _i32)
    m_acc = jnp.zeros((L,), _f32)
    xt_acc = jnp.zeros((L,), _f32)
    for j in range(RPW):
        row = RPW * w + j
        pltpu.sync_copy(x_hbm.at[row], row_v)
        U = 4
        def maxbody(i, accs):
            base = i * (L * U)
            return tuple(
                jnp.maximum(accs[u], row_v[pl.ds(base + u * L, L)])
                for u in range(U))
        accs = lax.fori_loop(0, NV // U, maxbody,
                             tuple(_negvec() for _ in range(U)))
        acc = accs[0]
        for u in range(1, U):
            acc = jnp.maximum(acc, accs[u])
        mrow = jnp.max(acc)
        c = _collect(row_v, cand_v, mrow - _f32(1.03125))
        def ext_cond(st):
            return st[3] == 0
        def ext_body(st):
            m0, s, k, done = st
            vs = _sweep_max(cand_v, c)
            mu = _sweep_count_remove(cand_v, c, vs)
            in_sup = ((m0 + 1).astype(_f32) * vs) > ((s + vs) - _f32(1.0))
            new_m0 = jnp.where(in_sup, m0 + mu, m0)
            new_s = jnp.where(in_sup, s + mu.astype(_f32) * vs, s)
            exhausted = (new_m0 >= c).astype(_i32)
            new_done = jnp.where(in_sup, exhausted, _i32(1))
            new_k = jnp.where(in_sup, new_m0, m0)
            return (new_m0, new_s, new_k, new_done)
        _, s, k, _ = lax.while_loop(
            ext_cond, ext_body, (_i32(0), _f32(0.0), _i32(1), _i32(0)))
        trow = jnp.max(plsc.load_gather(tgt_v, [jnp.full((L,), row, _i32)]))
        xtv = jnp.max(plsc.load_gather(row_v, [jnp.full((L,), trow, _i32)]))
        sel = lane == j
        tau_acc = jnp.where(sel, jnp.full((L,), s, _f32), tau_acc)
        k_acc = jnp.where(sel, jnp.full((L,), k, _i32), k_acc)
        m_acc = jnp.where(sel, jnp.full((L,), mrow, _f32), m_acc)
        xt_acc = jnp.where(sel, jnp.full((L,), xtv, _f32), xt_acc)
    tau_s[...] = tau_acc
    k_s[...] = k_acc
    m_s[...] = m_acc
    xt_s[...] = xt_acc
    pltpu.sync_copy(tau_s, tau_hbm.at[w])
    pltpu.sync_copy(k_s, k_hbm.at[w])
    pltpu.sync_copy(m_s, m_hbm.at[w])
    pltpu.sync_copy(xt_s, xt_hbm.at[w])


def _stage2_body(x_hbm, k_hbm, m_hbm, sq_hbm,
                 row_v, cand_v, kk_v, m8_v, sq_v):
    w = _wid()
    pltpu.sync_copy(k_hbm, kk_v)
    pltpu.sync_copy(m_hbm.at[w], m8_v)
    def tmax(i, a):
        return jnp.maximum(a, kk_v[i])
    tacc = lax.fori_loop(0, NW, tmax, jnp.full((L,), 1, _i32))
    tmaxk = jnp.minimum(jnp.max(tacc), _i32(TCAP))
    for j in range(RPW):
        row = RPW * w + j
        pltpu.sync_copy(x_hbm.at[row], row_v)
        mrow = jnp.max(plsc.load_gather(m8_v, [jnp.full((L,), j, _i32)]))
        c0 = _collect(row_v, cand_v, mrow - _f32(1.03125))
        def coll_cond(st):
            return st[1] < tmaxk
        def coll_body(st):
            delta, _ = st
            nd = jnp.where(delta < _f32(16.0), delta * _f32(1.5),
                           delta * _f32(256.0))
            return (nd, _collect(row_v, cand_v, mrow - nd))
        _, c = lax.while_loop(coll_cond, coll_body, (_f32(1.03125), c0))
        for b in range(TCAP // L):
            sq_v[pl.ds(b * L, L)] = jnp.zeros((L,), _f32)
        iota = lax.iota(_i32, L)
        def ext_cond(st):
            return st[2] == 0
        def ext_body(st):
            m0, s2, done = st
            vs = _sweep_max(cand_v, c)
            mu = _sweep_count_remove(cand_v, c, vs)
            vs2 = vs * vs
            lim = jnp.minimum(m0 + mu, tmaxk)
            nb = lax.shift_right_arithmetic(mu + (L - 1), 4)
            def fill(b, carry):
                idx = m0 + b * L + iota
                mask = idx < lim
                vals = s2 + (idx - m0 + 1).astype(_f32) * vs2
                plsc.store_scatter(sq_v, [idx], vals, mask=mask)
                return carry
            lax.fori_loop(0, nb, fill, _i32(0))
            new_m0 = m0 + mu
            new_s2 = s2 + mu.astype(_f32) * vs2
            return (new_m0, new_s2, (new_m0 >= tmaxk).astype(_i32))
        lax.while_loop(ext_cond, ext_body, (_i32(0), _f32(0.0), _i32(0)))
        pltpu.sync_copy(sq_v, sq_hbm.at[row])


@functools.cache
def _build_stage1():
    mesh = plsc.VectorSubcoreMesh(core_axis_name="c", subcore_axis_name="s")
    return pl.kernel(
        _stage1_body,
        out_type=(
            jax.ShapeDtypeStruct((NW, L), _f32),
            jax.ShapeDtypeStruct((NW, L), _i32),
            jax.ShapeDtypeStruct((NW, L), _f32),
            jax.ShapeDtypeStruct((NW, L), _f32),
        ),
        mesh=mesh,
        compiler_params=pltpu.CompilerParams(needs_layout_passes=False),
        scratch_types=[
            pltpu.VMEM((D,), _f32),
            pltpu.VMEM((D + L,), _f32),
            pltpu.VMEM((N,), _i32),
            pltpu.VMEM((L,), _f32),
            pltpu.VMEM((L,), _i32),
            pltpu.VMEM((L,), _f32),
            pltpu.VMEM((L,), _f32),
        ],
    )


@functools.cache
def _build_stage2():
    mesh = plsc.VectorSubcoreMesh(core_axis_name="c", subcore_axis_name="s")
    return pl.kernel(
        _stage2_body,
        out_type=jax.ShapeDtypeStruct((N, TCAP), _f32),
        mesh=mesh,
        compiler_params=pltpu.CompilerParams(needs_layout_passes=False),
        scratch_types=[
            pltpu.VMEM((D,), _f32),
            pltpu.VMEM((D + L,), _f32),
            pltpu.VMEM((NW, L), _i32),
            pltpu.VMEM((L,), _f32),
            pltpu.VMEM((TCAP,), _f32),
        ],
    )


def _combine_body(sq_ref, k_ref, s_ref, xt_ref, o_ref):
    sq = sq_ref[...]
    csum = jnp.sum(sq, axis=0)
    k = k_ref[...]
    kf = k.astype(_f32)
    tau = (s_ref[...] - _f32(1.0)) / kf
    xt = xt_ref[...]
    onehot = lax.broadcasted_iota(_i32, (N, TCAP), 1) == (k - 1)
    g = jnp.sum(jnp.where(onehot, csum[None, :], _f32(0.0)), axis=1,
                keepdims=True)
    term = g - _f32(float(N)) * tau * tau * kf
    o_ref[...] = -xt + (_f32(1.0) + term) * _f32(0.5)


@functools.cache
def _build_combine():
    return pl.pallas_call(
        _combine_body,
        out_shape=jax.ShapeDtypeStruct((N, 1), _f32),
    )


def kernel(X, target, proj_args):
    s8, k8, m8, xt8 = _build_stage1()(X, target)
    sqcs = _build_stage2()(X, k8, m8)
    s = s8[:, :RPW].reshape(N, 1)
    k = k8[:, :RPW].reshape(N, 1)
    xt = xt8[:, :RPW].reshape(N, 1)
    loss = _build_combine()(sqcs, k, s, xt)
    return loss.reshape(N)
